# baseline (speedup 1.0000x reference)
import functools

import jax
import jax.numpy as jnp
from jax import lax
from jax.experimental import pallas as pl
from jax.experimental.pallas import tpu as pltpu
from jax.experimental.pallas import tpu_sc as plsc


def _ymm_body(x_ref, w_ref, dinv_ref, y_ref):
    y = jnp.dot(x_ref[0], w_ref[...], preferred_element_type=jnp.float32)
    y_ref[0] = y * dinv_ref[...]


def _compute_y(xt, W1, dinv2):
    T, N, D = xt.shape
    H = W1.shape[1]
    BN = 1000
    grid = (T, N // BN)
    return pl.pallas_call(
        _ymm_body,
        grid=grid,
        in_specs=[
            pl.BlockSpec((1, BN, D), lambda t, nb: (t, nb, 0)),
            pl.BlockSpec((D, H), lambda t, nb: (0, 0)),
            pl.BlockSpec((BN, 1), lambda t, nb: (nb, 0)),
        ],
        out_specs=pl.BlockSpec((1, BN, H), lambda t, nb: (t, nb, 0)),
        out_shape=jax.ShapeDtypeStruct((T, N, H), jnp.float32),
    )(xt, W1, dinv2)


_NC, _NS = 2, 16
_NW = _NC * _NS
_CH = 256


def _sc_scatter_call(y2, gidx, dstr, zrows, N, T, H):
    NCH = dstr.shape[1] // 2
    NHF = NCH // 2
    NWR = 10
    RPW = N // NWR
    ZR = zrows.shape[0]

    @functools.partial(
        pl.kernel,
        mesh=plsc.VectorSubcoreMesh(core_axis_name="c", subcore_axis_name="s"),
        out_type=jax.ShapeDtypeStruct((_NC, T, N, H), jnp.float32),
        scratch_types=[
            pltpu.VMEM((NHF * _CH,), jnp.int32),
            pltpu.VMEM((2 * NCH, _CH // 2), jnp.int32),
            pltpu.VMEM((_CH, H), jnp.float32),
            pltpu.VMEM_SHARED((N + 8, H), jnp.float32),
            pltpu.SemaphoreType.DMA,
        ],
    )
    def body(y2_hbm, gidx_hbm, dstr_hbm, zrows_hbm, out_hbm,
             gidxv, dstv, rows2, acc_sh, sem_a):
        c = lax.axis_index("c")
        s = lax.axis_index("s")
        wid = c * _NS + s
        pltpu.sync_copy(dstr_hbm.at[wid], dstv)
        for t in range(T):
            @pl.when(s < NWR)
            def _():
                for z in range(RPW // ZR):
                    pltpu.sync_copy(zrows_hbm,
                                    acc_sh.at[pl.ds(s * RPW + z * ZR, ZR)])
            plsc.subcore_barrier()

            for half in range(2):
                pltpu.sync_copy(gidx_hbm.at[t, wid, half], gidxv)
                base = half * NHF

                def chunk(j, carry):
                    off = pl.multiple_of(j * _CH, _CH)
                    pltpu.async_copy(y2_hbm.at[gidxv.at[pl.ds(off, _CH)]],
                                     rows2, sem_a).wait()
                    jd = 2 * (base + j)
                    pltpu.sync_copy(rows2.at[pl.ds(0, _CH // 2)],
                                    acc_sh.at[dstv.at[jd]], add=True)
                    pltpu.sync_copy(rows2.at[pl.ds(_CH // 2, _CH // 2)],
                                    acc_sh.at[dstv.at[jd + 1]], add=True)
                    return carry

                lax.fori_loop(0, NHF, chunk, 0)
            plsc.subcore_barrier()

            @pl.when(s < NWR)
            def _():
                pltpu.sync_copy(acc_sh.at[pl.ds(s * RPW, RPW)],
                                out_hbm.at[c, t, pl.ds(s * RPW, RPW)])

    return body(y2, gidx, dstr, zrows)


def _lstm_body(aggp_ref, y_ref, dinv_ref, b1_ref, wih_ref, whh_ref, bsum_ref,
               wo_ref, bo_ref, out_ref):
    T, BN, H = y_ref.shape
    dinv = dinv_ref[...]
    h = jnp.zeros((BN, H), jnp.float32)
    c = jnp.zeros((BN, H), jnp.float32)
    for t in range(T):
        s = y_ref[t] + aggp_ref[0, t] + aggp_ref[1, t]
        a = jnp.maximum(s * dinv + b1_ref[...], 0.0)
        gates = (jnp.dot(a, wih_ref[...], preferred_element_type=jnp.float32)
                 + jnp.dot(h, whh_ref[...], preferred_element_type=jnp.float32)
                 + bsum_ref[...])
        i_g = jax.nn.sigmoid(gates[:, :H])
        f_g = jax.nn.sigmoid(gates[:, H:2 * H])
        g_g = jnp.tanh(gates[:, 2 * H:3 * H])
        o_g = jax.nn.sigmoid(gates[:, 3 * H:])
        c = f_g * c + i_g * g_g
        h = o_g * jnp.tanh(c)
    out_ref[...] = (jnp.dot(h, wo_ref[...], preferred_element_type=jnp.float32)
                    + bo_ref[...])


def _lstm(aggp, y, dinv2, b1r, wihT, whhT, bsum, Wo, bor):
    P, T, N, H = aggp.shape
    O = Wo.shape[1]
    BN = 1000
    grid = (N // BN,)
    return pl.pallas_call(
        _lstm_body,
        grid=grid,
        in_specs=[
            pl.BlockSpec((P, T, BN, H), lambda nb: (0, 0, nb, 0)),
            pl.BlockSpec((T, BN, H), lambda nb: (0, nb, 0)),
            pl.BlockSpec((BN, 1), lambda nb: (nb, 0)),
            pl.BlockSpec((1, H), lambda nb: (0, 0)),
            pl.BlockSpec((H, 4 * H), lambda nb: (0, 0)),
            pl.BlockSpec((H, 4 * H), lambda nb: (0, 0)),
            pl.BlockSpec((1, 4 * H), lambda nb: (0, 0)),
            pl.BlockSpec((H, O), lambda nb: (0, 0)),
            pl.BlockSpec((1, O), lambda nb: (0, 0)),
        ],
        out_specs=pl.BlockSpec((BN, O), lambda nb: (nb, 0)),
        out_shape=jax.ShapeDtypeStruct((N, O), jnp.float32),
    )(aggp, y, dinv2, b1r, wihT, whhT, bsum, Wo, bor)


def kernel(x, edge_index, W1, b1, W_ih, W_hh, b_ih, b_hh, Wo, bo):
    N, T, D = x.shape
    H = W1.shape[1]
    src = edge_index[0]
    dst = edge_index[1]

    deg = jnp.zeros((N,), jnp.float32).at[dst].add(1.0) + 1.0
    dinv = jax.lax.rsqrt(deg)
    dinv2 = dinv[:, None]

    xt = jnp.transpose(x, (1, 0, 2))
    y = _compute_y(xt, W1, dinv2)

    E = src.shape[0]
    EPG = 2 * _NW * _CH
    EP = ((E + EPG - 1) // EPG) * EPG
    NCH = EP // (_NW * _CH)
    pad = EP - E
    srcp = jnp.concatenate([src, jnp.zeros((pad,), jnp.int32)])
    dstp = jnp.concatenate([dst, jnp.full((pad,), N, jnp.int32)])
    gidx = (srcp[None, :] + (jnp.arange(T, dtype=jnp.int32) * N)[:, None])
    gidx = gidx.reshape(T, _NW, 2, (NCH // 2) * _CH)
    dstr = dstp.reshape(_NW, 2 * NCH, _CH // 2)
    zrows = jnp.zeros((200, H), jnp.float32)
    aggp = _sc_scatter_call(y.reshape(T * N, H), gidx, dstr, zrows, N, T, H)

    b1r = b1[None, :]
    bsum = (b_ih + b_hh)[None, :]
    bor = bo[None, :]
    return _lstm(aggp, y, dinv2, b1r, W_ih.T, W_hh.T, bsum, Wo, bor)


# speedup vs baseline: 1.3839x; 1.3839x over previous
import functools

import jax
import jax.numpy as jnp
from jax import lax
from jax.experimental import pallas as pl
from jax.experimental.pallas import tpu as pltpu
from jax.experimental.pallas import tpu_sc as plsc


def _ymm_body(x_ref, w_ref, dinv_ref, y_ref):
    y = jnp.dot(x_ref[0], w_ref[...], preferred_element_type=jnp.float32)
    y_ref[0] = y * dinv_ref[...]


def _compute_y(xt, W1, dinv2):
    T, N, D = xt.shape
    H = W1.shape[1]
    BN = 1000
    grid = (T, N // BN)
    return pl.pallas_call(
        _ymm_body,
        grid=grid,
        in_specs=[
            pl.BlockSpec((1, BN, D), lambda t, nb: (t, nb, 0)),
            pl.BlockSpec((D, H), lambda t, nb: (0, 0)),
            pl.BlockSpec((BN, 1), lambda t, nb: (nb, 0)),
        ],
        out_specs=pl.BlockSpec((1, BN, H), lambda t, nb: (t, nb, 0)),
        out_shape=jax.ShapeDtypeStruct((T, N, H), jnp.float32),
    )(xt, W1, dinv2)


_NC, _NS = 2, 16
_NW = _NC * _NS
_CH = 128


def _sc_scatter_call(y2, gidx, dstr, zrows, N, T, H):
    NCH = dstr.shape[1]
    NWR = 10
    RPW = N // NWR
    ZR = zrows.shape[0]

    @functools.partial(
        pl.kernel,
        mesh=plsc.VectorSubcoreMesh(core_axis_name="c", subcore_axis_name="s"),
        out_type=jax.ShapeDtypeStruct((_NC, T, N, H), jnp.float32),
        scratch_types=[
            pltpu.VMEM((NCH, _CH), jnp.int32),
            pltpu.VMEM((NCH, _CH), jnp.int32),
            pltpu.VMEM((_CH, H), jnp.float32),
            pltpu.VMEM_SHARED((N + 8, H), jnp.float32),
            pltpu.SemaphoreType.DMA,
        ],
    )
    def body(y2_hbm, gidx_hbm, dstr_hbm, zrows_hbm, out_hbm,
             gidxv, dstv, rows, acc_sh, sem):
        c = lax.axis_index("c")
        s = lax.axis_index("s")
        wid = c * _NS + s
        pltpu.sync_copy(dstr_hbm.at[wid], dstv)
        for t in range(T):
            @pl.when(s < NWR)
            def _():
                for z in range(RPW // ZR):
                    pltpu.sync_copy(zrows_hbm,
                                    acc_sh.at[pl.ds(s * RPW + z * ZR, ZR)])
            pltpu.sync_copy(gidx_hbm.at[t, wid], gidxv)
            plsc.subcore_barrier()

            def chunk(j, carry):
                pltpu.async_copy(y2_hbm.at[gidxv.at[j]], rows, sem).wait()
                pltpu.sync_copy(rows, acc_sh.at[dstv.at[j]], add=True)
                return carry

            lax.fori_loop(0, NCH, chunk, 0)
            plsc.subcore_barrier()

            @pl.when(s < NWR)
            def _():
                pltpu.sync_copy(acc_sh.at[pl.ds(s * RPW, RPW)],
                                out_hbm.at[c, t, pl.ds(s * RPW, RPW)])

    return body(y2, gidx, dstr, zrows)


def _lstm_body(aggp_ref, y_ref, dinv_ref, b1_ref, wih_ref, whh_ref, bsum_ref,
               wo_ref, bo_ref, out_ref):
    T, BN, H = y_ref.shape
    dinv = dinv_ref[...]
    h = jnp.zeros((BN, H), jnp.float32)
    c = jnp.zeros((BN, H), jnp.float32)
    for t in range(T):
        s = y_ref[t] + aggp_ref[0, t] + aggp_ref[1, t]
        a = jnp.maximum(s * dinv + b1_ref[...], 0.0)
        gates = (jnp.dot(a, wih_ref[...], preferred_element_type=jnp.float32)
                 + jnp.dot(h, whh_ref[...], preferred_element_type=jnp.float32)
                 + bsum_ref[...])
        i_g = jax.nn.sigmoid(gates[:, :H])
        f_g = jax.nn.sigmoid(gates[:, H:2 * H])
        g_g = jnp.tanh(gates[:, 2 * H:3 * H])
        o_g = jax.nn.sigmoid(gates[:, 3 * H:])
        c = f_g * c + i_g * g_g
        h = o_g * jnp.tanh(c)
    out_ref[...] = (jnp.dot(h, wo_ref[...], preferred_element_type=jnp.float32)
                    + bo_ref[...])


def _lstm(aggp, y, dinv2, b1r, wihT, whhT, bsum, Wo, bor):
    P, T, N, H = aggp.shape
    O = Wo.shape[1]
    BN = 1000
    grid = (N // BN,)
    return pl.pallas_call(
        _lstm_body,
        grid=grid,
        in_specs=[
            pl.BlockSpec((P, T, BN, H), lambda nb: (0, 0, nb, 0)),
            pl.BlockSpec((T, BN, H), lambda nb: (0, nb, 0)),
            pl.BlockSpec((BN, 1), lambda nb: (nb, 0)),
            pl.BlockSpec((1, H), lambda nb: (0, 0)),
            pl.BlockSpec((H, 4 * H), lambda nb: (0, 0)),
            pl.BlockSpec((H, 4 * H), lambda nb: (0, 0)),
            pl.BlockSpec((1, 4 * H), lambda nb: (0, 0)),
            pl.BlockSpec((H, O), lambda nb: (0, 0)),
            pl.BlockSpec((1, O), lambda nb: (0, 0)),
        ],
        out_specs=pl.BlockSpec((BN, O), lambda nb: (nb, 0)),
        out_shape=jax.ShapeDtypeStruct((N, O), jnp.float32),
    )(aggp, y, dinv2, b1r, wihT, whhT, bsum, Wo, bor)


def kernel(x, edge_index, W1, b1, W_ih, W_hh, b_ih, b_hh, Wo, bo):
    N, T, D = x.shape
    H = W1.shape[1]
    src = edge_index[0]
    dst = edge_index[1]

    deg = jnp.zeros((N,), jnp.float32).at[dst].add(1.0) + 1.0
    dinv = jax.lax.rsqrt(deg)
    dinv2 = dinv[:, None]

    xt = jnp.transpose(x, (1, 0, 2))
    y = _compute_y(xt, W1, dinv2)

    E = src.shape[0]
    EPG = _NW * _CH
    EP = ((E + EPG - 1) // EPG) * EPG
    NCH = EP // (_NW * _CH)
    pad = EP - E
    srcp = jnp.concatenate([src, jnp.zeros((pad,), jnp.int32)])
    dstp = jnp.concatenate([dst, jnp.full((pad,), N, jnp.int32)])
    gidx = (srcp[None, :] + (jnp.arange(T, dtype=jnp.int32) * N)[:, None])
    gidx = gidx.reshape(T, _NW, NCH, _CH)
    dstr = dstp.reshape(_NW, NCH, _CH)
    zrows = jnp.zeros((200, H), jnp.float32)
    aggp = _sc_scatter_call(y.reshape(T * N, H), gidx, dstr, zrows, N, T, H)

    b1r = b1[None, :]
    bsum = (b_ih + b_hh)[None, :]
    bor = bo[None, :]
    return _lstm(aggp, y, dinv2, b1r, W_ih.T, W_hh.T, bsum, Wo, bor)
